# baseline (speedup 1.0000x reference)
import functools

import jax
import jax.numpy as jnp
from jax import lax
from jax.experimental import pallas as pl
from jax.experimental.pallas import tpu as pltpu
from jax.experimental.pallas import tpu_sc as plsc

N = 10000
E = 160000
D = 256
LAYERS = 5
B = 512
NT = 12
NE = 8

HD = 128
NC = 2
NS = 16
EK = 128
RPT = N // NS
MAXR = 256
NBX = 128
SLOTS = NS * NBX * EK
RPA = 624
TAIL = N - RPA * NS
NPAD = N + 8
ZTAIL = NPAD - RPA * NS
NCODE = 9
TB = 1000
NBLK = N // TB
SPLIT = 3


def _sc_adj_body(h9_hbm, idxb_hbm, dstb_hbm, z_hbm, out_hbm, idx_src,
                 idx_dst, rows, acc):
    cid = lax.axis_index("c")
    sid = lax.axis_index("s")
    pltpu.sync_copy(idxb_hbm.at[sid], idx_src)
    pltpu.sync_copy(dstb_hbm.at[sid], idx_dst)
    pltpu.sync_copy(z_hbm.at[pl.ds(sid * RPA, RPA)],
                    acc.at[pl.ds(sid * RPA, RPA)])

    @pl.when(sid == NS - 1)
    def _():
        pltpu.sync_copy(z_hbm.at[pl.ds(NS * RPA, ZTAIL)],
                        acc.at[pl.ds(NS * RPA, ZTAIL)])

    plsc.subcore_barrier()

    def step(j, carry):
        pltpu.sync_copy(h9_hbm.at[cid].at[idx_src.at[j]], rows)
        pltpu.sync_copy(rows, acc.at[idx_dst.at[j]], add=True)
        return carry

    lax.fori_loop(0, NBX, step, 0)
    plsc.subcore_barrier()
    pltpu.sync_copy(acc.at[pl.ds(sid * RPA, RPA)],
                    out_hbm.at[cid].at[pl.ds(sid * RPA, RPA)])

    @pl.when(sid == NS - 1)
    def _():
        pltpu.sync_copy(acc.at[pl.ds(NS * RPA, TAIL)],
                        out_hbm.at[cid].at[pl.ds(NS * RPA, TAIL)])


@functools.lru_cache(maxsize=None)
def _make_sc_adj():
    return pl.kernel(
        _sc_adj_body,
        out_type=jax.ShapeDtypeStruct((NC, N, HD), jnp.float32),
        mesh=plsc.VectorSubcoreMesh(core_axis_name="c",
                                    subcore_axis_name="s",
                                    num_cores=NC, num_subcores=NS),
        scratch_types=[
            pltpu.VMEM((NBX, EK), jnp.int32),
            pltpu.VMEM((NBX, EK), jnp.int32),
            pltpu.VMEM((EK, HD), jnp.float32),
            pltpu.VMEM_SHARED((NPAD, HD), jnp.float32),
        ],
    )


def _emit_h9(y, comb_ref, sl_ref, out9_ref, hsl_ref):
    for c in range(NCODE):
        yc = y + comb_ref[c:c + 1, :]
        out9_ref[0, c] = yc[:, :HD]
        out9_ref[1, c] = yc[:, HD:]
    ysl = y + sl_ref[...]
    hsl_ref[0] = ysl[:, :HD]
    hsl_ref[1] = ysl[:, HD:]


def _tc_embed_body(x_ref, e1_ref, e2_ref, out_ref):
    x0 = x_ref[:, 0:1]
    x1 = x_ref[:, 1:2]
    h = jnp.zeros((TB, D), jnp.float32)
    for v in range(3):
        h = h + jnp.where(x0 == v, 1.0, 0.0) * e1_ref[v:v + 1, :]
        h = h + jnp.where(x1 == v, 1.0, 0.0) * e2_ref[v:v + 1, :]
    out_ref[...] = h


def _tc_emit9_body(h_ref, comb_ref, sl_ref, out9_ref, hsl_ref):
    _emit_h9(h_ref[...], comb_ref, sl_ref, out9_ref, hsl_ref)


def _tc_mlp_plain_body(aggr_ref, w1_ref, b1_ref, w2_ref, b2_ref, hn_ref):
    hm = jnp.maximum(
        jnp.dot(aggr_ref[...], w1_ref[...],
                preferred_element_type=jnp.float32) + b1_ref[...], 0.0)
    hn_ref[...] = (jnp.dot(hm, w2_ref[...],
                           preferred_element_type=jnp.float32)
                   + b2_ref[...])


def _tc_mlp_body(acc_ref, hsl_ref, w1_ref, b1_ref, w2_ref, b2_ref, hn_ref,
                 stat_ref, ssum, ssq):
    i = pl.program_id(0)
    aggr = (jnp.concatenate([acc_ref[0], acc_ref[1]], axis=1)
            + jnp.concatenate([hsl_ref[0], hsl_ref[1]], axis=1))
    hm = jnp.maximum(
        jnp.dot(aggr, w1_ref[...], preferred_element_type=jnp.float32)
        + b1_ref[...], 0.0)
    hn = (jnp.dot(hm, w2_ref[...], preferred_element_type=jnp.float32)
          + b2_ref[...])
    hn_ref[...] = hn
    s1 = jnp.sum(hn, axis=0, keepdims=True)
    s2 = jnp.sum(hn * hn, axis=0, keepdims=True)

    @pl.when(i == 0)
    def _():
        ssum[...] = s1
        ssq[...] = s2

    @pl.when(i > 0)
    def _():
        ssum[...] += s1
        ssq[...] += s2

    @pl.when(i == NBLK - 1)
    def _():
        m = ssum[...] / N
        v = ssq[...] / N - m * m
        stat_ref[0:1] = m
        stat_ref[1:2] = v


def _bn_y(hn_ref, stat_ref, g_ref, b_ref, relu):
    m = stat_ref[0:1]
    v = stat_ref[1:2]
    y = g_ref[...] * (hn_ref[...] - m) / jnp.sqrt(v + 1e-5) + b_ref[...]
    if relu:
        return jnp.maximum(y, 0.0)
    return y


def _tc_bnrelu9_body(hn_ref, stat_ref, g_ref, b_ref, comb_ref, sl_ref,
                     out9_ref, hsl_ref):
    y = _bn_y(hn_ref, stat_ref, g_ref, b_ref, True)
    _emit_h9(y, comb_ref, sl_ref, out9_ref, hsl_ref)


def _tc_bnlast_body(hn_ref, stat_ref, g_ref, b_ref, out_ref):
    y = _bn_y(hn_ref, stat_ref, g_ref, b_ref, False)
    out_ref[0] = y[:, :HD]
    out_ref[1] = y[:, HD:]


def _tc_head_body(h2_ref, batch_ref, gw1_ref, gb1_ref, bng_ref, bnb_ref,
                  gw2_ref, gb2_ref, cl_ref, ew_ref, eb_ref, pred_ref, gacc):
    i = pl.program_id(0)
    h_blk = jnp.concatenate([h2_ref[0], h2_ref[1]], axis=1)
    sb = jnp.where(
        batch_ref[...] == lax.broadcasted_iota(jnp.int32, (TB, B), 1),
        1.0, 0.0)
    part = lax.dot_general(sb, h_blk, (((0,), (0,)), ((), ())),
                           preferred_element_type=jnp.float32,
                           precision=lax.Precision.HIGHEST)

    @pl.when(i == 0)
    def _():
        gacc[...] = part

    @pl.when(i > 0)
    def _():
        gacc[...] += part

    @pl.when(i == NBLK - 1)
    def _():
        graph = gacc[...]
        g1 = (jnp.dot(graph, gw1_ref[...], preferred_element_type=jnp.float32)
              + gb1_ref[...])
        m = jnp.mean(g1, axis=0, keepdims=True)
        v = jnp.mean((g1 - m) ** 2, axis=0, keepdims=True)
        g1 = bng_ref[...] * (g1 - m) / jnp.sqrt(v + 1e-5) + bnb_ref[...]
        g1 = jnp.maximum(g1, 0.0)
        ge = (jnp.dot(g1, gw2_ref[...], preferred_element_type=jnp.float32)
              + gb2_ref[...])
        gn = ge / (jnp.sqrt(jnp.sum(ge * ge, axis=1, keepdims=True)) + 1e-6)
        cl = cl_ref[...]
        cn = cl / (jnp.sqrt(jnp.sum(cl * cl, axis=1, keepdims=True)) + 1e-6)
        logits = 10.0 * jnp.dot(gn, cn.T, preferred_element_type=jnp.float32)
        logits = logits - jnp.max(logits, axis=1, keepdims=True)
        p = jnp.exp(logits)
        assign = p / jnp.sum(p, axis=1, keepdims=True)
        eo = (jnp.dot(graph, ew_ref[...], preferred_element_type=jnp.float32)
              + eb_ref[...])
        j_e = lax.broadcasted_iota(jnp.int32, (NE, NE * NT), 1)
        e_row = lax.broadcasted_iota(jnp.int32, (NE, NE * NT), 0)
        rmat = jnp.where(j_e // NT == e_row, 1.0, 0.0)
        a2 = jnp.dot(assign, rmat, preferred_element_type=jnp.float32,
                     precision=lax.Precision.HIGHEST)
        j_t = lax.broadcasted_iota(jnp.int32, (NE * NT, NT), 0)
        t_col = lax.broadcasted_iota(jnp.int32, (NE * NT, NT), 1)
        smat = jnp.where(j_t % NT == t_col, 1.0, 0.0)
        pred_ref[...] = jnp.dot(eo * a2, smat,
                                preferred_element_type=jnp.float32,
                                precision=lax.Precision.HIGHEST)


def kernel(x, edge_index, edge_attr, batch, x_emb1, x_emb2, edge_emb1,
           edge_emb2, W1, b1, W2, b2, bn_g, bn_b, gate_W1, gate_b1,
           gate_bng, gate_bnb, gate_W2, gate_b2, cluster, experts_w,
           experts_b):
    f32 = jnp.float32
    src = edge_index[0].astype(jnp.int32)
    dst = edge_index[1].astype(jnp.int32)
    code = (edge_attr[:, 0] * 3 + edge_attr[:, 1]).astype(jnp.int32)

    order1 = jnp.argsort(dst, stable=True)
    dst_s = dst[order1]
    src_s = src[order1]
    code_s = code[order1]
    first = jnp.searchsorted(dst_s, dst_s, side='left').astype(jnp.int32)
    rank = jnp.minimum(jnp.arange(E, dtype=jnp.int32) - first, MAXR - 1)
    tile = dst_s // RPT
    g = tile * MAXR + rank
    cnts = jax.ops.segment_sum(jnp.ones((E,), jnp.int32), g,
                               num_segments=NS * MAXR)
    padded = ((cnts + EK - 1) // EK) * EK
    pc = padded.reshape(NS, MAXR)
    wo = (jnp.cumsum(pc, axis=1) - pc).reshape(-1)
    order2 = jnp.argsort(g, stable=True)
    g2 = g[order2]
    first2 = jnp.searchsorted(g2, g2, side='left').astype(jnp.int32)
    wg = jnp.arange(E, dtype=jnp.int32) - first2
    slot = tile[order2] * (NBX * EK) + wo[g2] + wg
    idx9_s = code_s * N + src_s
    base_idx = jnp.arange(SLOTS, dtype=jnp.int32) % N
    idxb = base_idx.at[slot].set(idx9_s[order2]).reshape(NS, NBX, EK)
    dpad = N + (jnp.arange(SLOTS, dtype=jnp.int32) % 8)
    dstb = dpad.at[slot].set(dst_s[order2]).reshape(NS, NBX, EK)
    zrow = jnp.zeros((NPAD, HD), f32)

    ci = jnp.arange(NCODE)
    comb = edge_emb1[:, ci // 3, :] + edge_emb2[:, ci % 3, :]
    sl = edge_emb1[:, 4, :] + edge_emb2[:, 0, :]
    e1 = x_emb1[:8]
    e2 = jnp.pad(x_emb2, ((0, 5), (0, 0)))

    def wspec(s):
        return pl.BlockSpec(s, lambda i: tuple(0 for _ in s))

    h9spec = pl.BlockSpec((NC, NCODE, TB, HD), lambda i: (0, 0, i, 0))
    h2spec = pl.BlockSpec((NC, TB, HD), lambda i: (0, i, 0))
    out9_shapes = [jax.ShapeDtypeStruct((NC, NCODE, N, HD), f32),
                   jax.ShapeDtypeStruct((NC, N, HD), f32)]

    hfull = pl.pallas_call(
        _tc_embed_body,
        grid=(NBLK,),
        in_specs=[
            pl.BlockSpec((TB, 2), lambda i: (i, 0)),
            wspec((8, D)), wspec((8, D)),
        ],
        out_specs=pl.BlockSpec((TB, D), lambda i: (i, 0)),
        out_shape=jax.ShapeDtypeStruct((N, D), f32),
    )(x.astype(jnp.int32), e1, e2)

    mlp_plain_call = pl.pallas_call(
        _tc_mlp_plain_body,
        grid=(NBLK,),
        in_specs=[
            pl.BlockSpec((TB, D), lambda i: (i, 0)),
            wspec((D, 2 * D)), wspec((1, 2 * D)),
            wspec((2 * D, D)), wspec((1, D)),
        ],
        out_specs=pl.BlockSpec((TB, D), lambda i: (i, 0)),
        out_shape=jax.ShapeDtypeStruct((N, D), f32),
    )

    loops = jnp.arange(N, dtype=edge_index.dtype)
    ei = jnp.concatenate([edge_index, jnp.stack([loops, loops])], axis=1)
    sl_attr = jnp.stack([jnp.full((N,), 4, dtype=edge_attr.dtype),
                         jnp.zeros((N,), dtype=edge_attr.dtype)], axis=1)
    ea = jnp.concatenate([edge_attr, sl_attr], axis=0)
    h = hfull
    for l in range(SPLIT):
        eemb = edge_emb1[l][ea[:, 0]] + edge_emb2[l][ea[:, 1]]
        msg = h[ei[0]] + eemb
        aggr = jax.ops.segment_sum(msg, ei[1], num_segments=N)
        hn = mlp_plain_call(aggr, W1[l], b1[l][None], W2[l], b2[l][None])
        m = hn.mean(axis=0)
        v = hn.var(axis=0)
        hn = bn_g[l] * (hn - m) / jnp.sqrt(v + 1e-5) + bn_b[l]
        h = jax.nn.relu(hn)

    h9, hsl = pl.pallas_call(
        _tc_emit9_body,
        grid=(NBLK,),
        in_specs=[
            pl.BlockSpec((TB, D), lambda i: (i, 0)),
            wspec((NCODE, D)), wspec((1, D)),
        ],
        out_specs=[h9spec, h2spec],
        out_shape=out9_shapes,
    )(h, comb[SPLIT], sl[SPLIT][None])

    mlp_call = pl.pallas_call(
        _tc_mlp_body,
        grid=(NBLK,),
        in_specs=[
            h2spec, h2spec,
            wspec((D, 2 * D)), wspec((1, 2 * D)),
            wspec((2 * D, D)), wspec((1, D)),
        ],
        out_specs=[
            pl.BlockSpec((TB, D), lambda i: (i, 0)),
            wspec((8, D)),
        ],
        out_shape=[
            jax.ShapeDtypeStruct((N, D), f32),
            jax.ShapeDtypeStruct((8, D), f32),
        ],
        scratch_shapes=[
            pltpu.VMEM((1, D), f32),
            pltpu.VMEM((1, D), f32),
        ],
    )

    h2 = None
    for l in range(SPLIT, LAYERS):
        acc2 = _make_sc_adj()(h9.reshape(NC, NCODE * N, HD), idxb, dstb,
                              zrow)
        hn, stat = mlp_call(acc2, hsl, W1[l], b1[l][None], W2[l],
                            b2[l][None])
        if l < LAYERS - 1:
            h9, hsl = pl.pallas_call(
                _tc_bnrelu9_body,
                grid=(NBLK,),
                in_specs=[
                    pl.BlockSpec((TB, D), lambda i: (i, 0)),
                    wspec((8, D)), wspec((1, D)), wspec((1, D)),
                    wspec((NCODE, D)), wspec((1, D)),
                ],
                out_specs=[h9spec, h2spec],
                out_shape=out9_shapes,
            )(hn, stat, bn_g[l][None], bn_b[l][None], comb[l + 1],
              sl[l + 1][None])
        else:
            h2 = pl.pallas_call(
                _tc_bnlast_body,
                grid=(NBLK,),
                in_specs=[
                    pl.BlockSpec((TB, D), lambda i: (i, 0)),
                    wspec((8, D)), wspec((1, D)), wspec((1, D)),
                ],
                out_specs=h2spec,
                out_shape=jax.ShapeDtypeStruct((NC, N, HD), f32),
            )(hn, stat, bn_g[l][None], bn_b[l][None])

    pred = pl.pallas_call(
        _tc_head_body,
        grid=(NBLK,),
        in_specs=[
            h2spec,
            pl.BlockSpec((TB, 1), lambda i: (i, 0)),
            wspec((D, D)), wspec((1, D)), wspec((1, D)), wspec((1, D)),
            wspec((D, D)), wspec((1, D)), wspec((NE, D)),
            wspec((D, NE * NT)), wspec((1, NE * NT)),
        ],
        out_specs=pl.BlockSpec((B, NT), lambda i: (0, 0)),
        out_shape=jax.ShapeDtypeStruct((B, NT), f32),
        scratch_shapes=[pltpu.VMEM((B, D), f32)],
    )(h2, batch.reshape(N, 1).astype(jnp.int32), gate_W1, gate_b1[None],
      gate_bng[None], gate_bnb[None], gate_W2, gate_b2[None], cluster,
      experts_w, experts_b[None])
    return pred
